# baseline (speedup 1.0000x reference)
import functools
import jax
import jax.numpy as jnp
from jax import lax
from jax.experimental import pallas as pl
from jax.experimental.pallas import tpu as pltpu
from jax.experimental.pallas import tpu_sc as plsc

NC = 2
NS = 16
NW = NC * NS
CH = 128
_SPLIT = (48, 32)


def _rsqrt16(d):
    ib = plsc.bitcast(d, jnp.int32)
    y = plsc.bitcast(jnp.int32(0x5F3759DF) - (ib >> 1), jnp.float32)
    for _ in range(4):
        y = y * (1.5 - 0.5 * d * y * y)
    return jnp.where(d > 0, y, 0.0)


def _sc_pre_body(n_pad, ept, src2_hbm, dst2_hbm, ew2_hbm, norm_hbm,
                 deg_sh, dis_sh, zbuf, zbuf2, dis_v, srcd_v, ewd_v,
                 srcn_v, dstn_v, ewn_v, nrm_v, sem):
    cid = lax.axis_index("c")
    sid = lax.axis_index("s")
    wid = cid * NS + sid
    rows_pt = n_pad // NS
    nch = ept // CH
    nchd = nch * NC

    for j in range(rows_pt // 16):
        zbuf[pl.ds(j * 16, 16)] = jnp.zeros((16,), jnp.float32)
    pltpu.sync_copy(zbuf, deg_sh.at[pl.ds(sid * rows_pt, rows_pt)])
    plsc.subcore_barrier()

    pltpu.sync_copy(src2_hbm.at[pl.ds(sid * nchd, nchd)], srcd_v)
    pltpu.sync_copy(ew2_hbm.at[pl.ds(sid * nchd, nchd)], ewd_v)

    def fire(k, _):
        pltpu.async_copy(ewd_v.at[k], deg_sh.at[srcd_v.at[k]], sem, add=True)
        return 0

    def drain(k, _):
        pltpu.make_async_copy(ewd_v.at[0], deg_sh.at[srcd_v.at[0]], sem).wait()
        return 0

    lax.fori_loop(0, nchd, fire, 0)
    lax.fori_loop(0, nchd, drain, 0)
    plsc.subcore_barrier()

    pltpu.sync_copy(deg_sh.at[pl.ds(sid * rows_pt, rows_pt)], zbuf)
    for j in range(rows_pt // 16):
        zbuf2[pl.ds(j * 16, 16)] = _rsqrt16(zbuf[pl.ds(j * 16, 16)])
    pltpu.sync_copy(zbuf2, dis_sh.at[pl.ds(sid * rows_pt, rows_pt)])
    plsc.subcore_barrier()
    pltpu.sync_copy(dis_sh, dis_v)

    pltpu.sync_copy(src2_hbm.at[pl.ds(wid * nch, nch)], srcn_v)
    pltpu.sync_copy(dst2_hbm.at[pl.ds(wid * nch, nch)], dstn_v)
    pltpu.sync_copy(ew2_hbm.at[pl.ds(wid * nch, nch)], ewn_v)

    def nchunk(k, _):
        for j in range(CH // 16):
            sl = pl.ds(j * 16, 16)
            a = plsc.load_gather(dis_v, [srcn_v[k, sl]])
            b = plsc.load_gather(dis_v, [dstn_v[k, sl]])
            nrm_v[k, sl] = -(a * ewn_v[k, sl]) * b
        return 0

    lax.fori_loop(0, nch, nchunk, 0)
    pltpu.sync_copy(nrm_v, norm_hbm.at[pl.ds(wid * nch, nch)])


def _sc_pre(src2, dst2, ew2, n_pad, nch_tot):
    nrows = src2.shape[0]
    nch = nch_tot // NW
    ept = nch * CH
    mesh = plsc.VectorSubcoreMesh(core_axis_name="c", subcore_axis_name="s")
    kfn = pl.kernel(
        functools.partial(_sc_pre_body, n_pad, ept),
        out_type=jax.ShapeDtypeStruct((nrows, CH), jnp.float32),
        mesh=mesh,
        compiler_params=pltpu.CompilerParams(needs_layout_passes=False),
        scratch_types=[
            pltpu.VMEM_SHARED((n_pad,), jnp.float32),
            pltpu.VMEM_SHARED((n_pad,), jnp.float32),
            pltpu.VMEM((n_pad // NS,), jnp.float32),
            pltpu.VMEM((n_pad // NS,), jnp.float32),
            pltpu.VMEM((n_pad,), jnp.float32),
            pltpu.VMEM((nch * NC, CH), jnp.int32),
            pltpu.VMEM((nch * NC, CH), jnp.float32),
            pltpu.VMEM((nch, CH), jnp.int32),
            pltpu.VMEM((nch, CH), jnp.int32),
            pltpu.VMEM((nch, CH), jnp.float32),
            pltpu.VMEM((nch, CH), jnp.float32),
            pltpu.SemaphoreType.DMA,
        ],
    )
    return kfn(src2, dst2, ew2)


def _sc_prop_body(n_pad, dc, n0, n1, bases, y_hbm, src2_hbm, dst2_hbm,
                  nrm2_hbm, out_hbm, acc_sh, rows_a, rows_b, src2_v, dst2_v,
                  nrm_v, shi_a, shi_a2, shi_b, shi_b2,
                  sem_a, sem_a2, sem_b, sem_b2):
    cid = lax.axis_index("c")
    sid = lax.axis_index("s")
    rows_pt = n_pad // NS
    row0 = jnp.where(cid == 0, sid * n0, NS * n0 + sid * n1)
    nsc_self = jnp.where(cid == 0, n0 // 8, n1 // 8)

    def zrow(e, _):
        for j in range(dc // 16):
            rows_a[e, pl.ds(j * 16, 16)] = jnp.zeros((16,), jnp.float32)
        return 0

    for p, gb in enumerate(bases):
        dual = len(gb) == 2

        def issue(k, base, rows, shi, sem, add=False):
            if base == 0:
                pltpu.async_copy(y_hbm.at[src2_v.at[k]], rows, sem, add=add)
            else:
                for j in range(CH // 16):
                    sl = pl.ds(j * 16, 16)
                    shi[sl] = src2_v[k, sl] + base
                pltpu.async_copy(y_hbm.at[shi], rows, sem, add=add)

        def wait(rows, sem):
            pltpu.make_async_copy(y_hbm.at[src2_v.at[0]], rows, sem).wait()

        def scale_scatter(k, rows):
            @plsc.parallel_loop(0, CH, unroll=4)
            def scale(e):
                nv = plsc.load_gather(
                    nrm_v, [jnp.zeros((16,), jnp.int32) + k,
                            jnp.zeros((16,), jnp.int32) + e])
                for j in range(dc // 16):
                    sl = pl.ds(j * 16, 16)
                    rows[e, sl] = rows[e, sl] * nv

            pltpu.sync_copy(rows, acc_sh.at[dst2_v.at[k]], add=True)

        lax.fori_loop(0, CH, zrow, 0)
        off = 0
        while off < rows_pt:
            sz = min(CH, rows_pt - off)
            pltpu.sync_copy(rows_a.at[pl.ds(0, sz)],
                            acc_sh.at[pl.ds(sid * rows_pt + off, sz)])
            off += sz
        plsc.subcore_barrier()

        def super_chunk(q, _):
            srow = row0 + q * 8
            pltpu.sync_copy(src2_hbm.at[pl.ds(srow, 8)], src2_v)
            pltpu.sync_copy(dst2_hbm.at[pl.ds(srow, 8)], dst2_v)
            pltpu.sync_copy(nrm2_hbm.at[pl.ds(srow, 8)], nrm_v)
            issue(0, gb[0], rows_a, shi_a, sem_a)
            for gg in range(4):
                k0 = 2 * gg
                wait(rows_a, sem_a)
                if dual:
                    issue(k0, gb[1], rows_a, shi_a2, sem_a2, add=True)
                issue(k0 + 1, gb[0], rows_b, shi_b, sem_b)
                if dual:
                    wait(rows_a, sem_a2)
                scale_scatter(k0, rows_a)
                wait(rows_b, sem_b)
                if dual:
                    issue(k0 + 1, gb[1], rows_b, shi_b2, sem_b2, add=True)
                if gg < 3:
                    issue(k0 + 2, gb[0], rows_a, shi_a, sem_a)
                if dual:
                    wait(rows_b, sem_b2)
                scale_scatter(k0 + 1, rows_b)
            return 0

        lax.fori_loop(0, nsc_self, super_chunk, 0)
        plsc.subcore_barrier()

        orow = (p * NC + cid) * n_pad + sid * rows_pt
        off = 0
        while off < rows_pt:
            sz = min(CH, rows_pt - off)
            pltpu.sync_copy(acc_sh.at[pl.ds(sid * rows_pt + off, sz)],
                            rows_a.at[pl.ds(0, sz)])
            pltpu.sync_copy(rows_a.at[pl.ds(0, sz)],
                            out_hbm.at[pl.ds(orow + off, sz)])
            off += sz


def _sc_prop(y, src2, dst2, norm2, n0, n1, bases=((0,),)):
    n_rows_y, dc = y.shape
    npass = len(bases)
    n_pad = NS * 640
    nmax = max(n0, n1)
    assert n0 % 8 == 0 and n1 % 8 == 0
    mesh = plsc.VectorSubcoreMesh(core_axis_name="c", subcore_axis_name="s")
    kfn = pl.kernel(
        functools.partial(_sc_prop_body, n_pad, dc, n0, n1, bases),
        out_type=jax.ShapeDtypeStruct((npass * NC * n_pad, dc), jnp.float32),
        mesh=mesh,
        compiler_params=pltpu.CompilerParams(needs_layout_passes=False),
        scratch_types=[
            pltpu.VMEM_SHARED((n_pad, dc), jnp.float32),
            pltpu.VMEM((CH, dc), jnp.float32),
            pltpu.VMEM((CH, dc), jnp.float32),
            pltpu.VMEM((8, CH), jnp.int32),
            pltpu.VMEM((8, CH), jnp.int32),
            pltpu.VMEM((8, CH), jnp.float32),
            pltpu.VMEM((CH,), jnp.int32),
            pltpu.VMEM((CH,), jnp.int32),
            pltpu.VMEM((CH,), jnp.int32),
            pltpu.VMEM((CH,), jnp.int32),
            pltpu.SemaphoreType.DMA,
            pltpu.SemaphoreType.DMA,
            pltpu.SemaphoreType.DMA,
            pltpu.SemaphoreType.DMA,
        ],
    )
    return kfn(y, src2, dst2, norm2)


def _l1pre_body(x_ref, p1a_ref, p1b_ref, p2a_ref, p2b_ref, w_ref, b_ref,
                pre_ref, st_ref):
    xc = jnp.concatenate([x_ref[...], p1a_ref[...] + p1b_ref[...],
                          p2a_ref[...] + p2b_ref[...]], axis=1)
    pre = jnp.dot(xc, w_ref[...], preferred_element_type=jnp.float32) + b_ref[...]
    pre_ref[...] = pre

    @pl.when(pl.program_id(0) == 0)
    def _():
        st_ref[...] = jnp.zeros_like(st_ref)

    st_ref[...] += jnp.stack([jnp.sum(pre, axis=0), jnp.sum(pre * pre, axis=0)])


def _tc_l1pre(x, p1a, p1b, p2a, p2b, w384, bias1, bm=1000):
    n = x.shape[0]
    f = w384.shape[1]
    return pl.pallas_call(
        _l1pre_body,
        out_shape=[jax.ShapeDtypeStruct((n, f), jnp.float32),
                   jax.ShapeDtypeStruct((2, f), jnp.float32)],
        grid=(n // bm,),
        in_specs=[pl.BlockSpec((bm, 128), lambda i: (i, 0)),
                  pl.BlockSpec((bm, 128), lambda i: (i, 0)),
                  pl.BlockSpec((bm, 128), lambda i: (i, 0)),
                  pl.BlockSpec((bm, 128), lambda i: (i, 0)),
                  pl.BlockSpec((bm, 128), lambda i: (i, 0)),
                  pl.BlockSpec((384, f), lambda i: (0, 0)),
                  pl.BlockSpec((f,), lambda i: (0,))],
        out_specs=[pl.BlockSpec((bm, f), lambda i: (i, 0)),
                   pl.BlockSpec((2, f), lambda i: (0, 0))],
    )(x, p1a, p1b, p2a, p2b, w384, bias1)


def _l1post_body(n_rows, pre_ref, st_ref, g_ref, bb_ref, wu1_ref, wu2_ref,
                 wu3_ref, whb_ref, b2_ref, u1_ref, u2_ref, u3_ref, hb_ref):
    m = st_ref[0, :] / n_rows
    var = st_ref[1, :] / n_rows - m * m
    scale = g_ref[...] * lax.rsqrt(var + 1e-5)
    shift = bb_ref[...] - m * scale
    h = jnp.maximum(pre_ref[...] * scale + shift, 0.0)
    u1_ref[...] = jnp.dot(h, wu1_ref[...], preferred_element_type=jnp.float32)
    u2_ref[...] = jnp.dot(h, wu2_ref[...], preferred_element_type=jnp.float32)
    u3_ref[...] = jnp.dot(h, wu3_ref[...], preferred_element_type=jnp.float32)
    hb_ref[...] = jnp.dot(h, whb_ref[...], preferred_element_type=jnp.float32) + b2_ref[...]


def _tc_l1post(pre, stats, g, b, wu1, wu2, wu3, whb, bias2, bm=1000):
    n, f = pre.shape
    du = wu1.shape[1]
    return pl.pallas_call(
        functools.partial(_l1post_body, float(n)),
        out_shape=[jax.ShapeDtypeStruct((n, du), jnp.float32),
                   jax.ShapeDtypeStruct((n, du), jnp.float32),
                   jax.ShapeDtypeStruct((n, du), jnp.float32),
                   jax.ShapeDtypeStruct((n, 300), jnp.float32)],
        grid=(n // bm,),
        in_specs=[pl.BlockSpec((bm, f), lambda i: (i, 0)),
                  pl.BlockSpec((2, f), lambda i: (0, 0)),
                  pl.BlockSpec((f,), lambda i: (0,)),
                  pl.BlockSpec((f,), lambda i: (0,)),
                  pl.BlockSpec((f, du), lambda i: (0, 0)),
                  pl.BlockSpec((f, du), lambda i: (0, 0)),
                  pl.BlockSpec((f, du), lambda i: (0, 0)),
                  pl.BlockSpec((f, 300), lambda i: (0, 0)),
                  pl.BlockSpec((300,), lambda i: (0,))],
        out_specs=[pl.BlockSpec((bm, du), lambda i: (i, 0)),
                   pl.BlockSpec((bm, du), lambda i: (i, 0)),
                   pl.BlockSpec((bm, du), lambda i: (i, 0)),
                   pl.BlockSpec((bm, 300), lambda i: (i, 0))],
    )(pre, stats, g, b, wu1, wu2, wu3, whb, bias2)


def _fpre_body(hb_ref, v1a_ref, v1b_ref, v2a_ref, v2b_ref, za_ref, zb_ref,
               o_ref, st_ref):
    bm = hb_ref.shape[0]
    v1 = (v1a_ref[...] + v1b_ref[...])[:, :100]
    v23 = (v2a_ref[...] + v2b_ref[...] + 2.0 * (za_ref[...] + zb_ref[...]))[:, :100]
    add = jnp.concatenate([jnp.zeros((bm, 100), jnp.float32), v1, v23], axis=1)
    o = hb_ref[...] + add
    o_ref[...] = o

    @pl.when(pl.program_id(0) == 0)
    def _():
        st_ref[...] = jnp.zeros_like(st_ref)

    st_ref[...] += jnp.stack([jnp.sum(o, axis=0), jnp.sum(o * o, axis=0)])


def _tc_fpre(hb, v1a, v1b, v2a, v2b, za, zb, bm=1000):
    n = hb.shape[0]
    du = v1a.shape[1]
    return pl.pallas_call(
        _fpre_body,
        out_shape=[jax.ShapeDtypeStruct((n, 300), jnp.float32),
                   jax.ShapeDtypeStruct((2, 300), jnp.float32)],
        grid=(n // bm,),
        in_specs=[pl.BlockSpec((bm, 300), lambda i: (i, 0))] +
                 [pl.BlockSpec((bm, du), lambda i: (i, 0))] * 6,
        out_specs=[pl.BlockSpec((bm, 300), lambda i: (i, 0)),
                   pl.BlockSpec((2, 300), lambda i: (0, 0))],
    )(hb, v1a, v1b, v2a, v2b, za, zb)


def _fpost_body(n_rows, o2_ref, st_ref, g_ref, bb_ref, lw_ref, lb_ref, o_ref):
    m = st_ref[0, :] / n_rows
    var = st_ref[1, :] / n_rows - m * m
    scale = g_ref[...] * lax.rsqrt(var + 1e-5)
    shift = bb_ref[...] - m * scale
    h2 = o2_ref[...] * scale + shift
    logits = jnp.dot(h2, lw_ref[...], preferred_element_type=jnp.float32) + lb_ref[...]
    mx = jnp.max(logits, axis=1, keepdims=True)
    lse = jnp.log(jnp.sum(jnp.exp(logits - mx), axis=1, keepdims=True)) + mx
    o_ref[...] = logits - lse


def _tc_fpost(out2, stats, g, b, lw, lb, bm=1000):
    n = out2.shape[0]
    k = lw.shape[1]
    return pl.pallas_call(
        functools.partial(_fpost_body, float(n)),
        out_shape=jax.ShapeDtypeStruct((n, k), jnp.float32),
        grid=(n // bm,),
        in_specs=[pl.BlockSpec((bm, 300), lambda i: (i, 0)),
                  pl.BlockSpec((2, 300), lambda i: (0, 0)),
                  pl.BlockSpec((300,), lambda i: (0,)),
                  pl.BlockSpec((300,), lambda i: (0,)),
                  pl.BlockSpec((300, k), lambda i: (0, 0)),
                  pl.BlockSpec((k,), lambda i: (0,))],
        out_specs=pl.BlockSpec((bm, k), lambda i: (i, 0)),
    )(out2, stats, g, b, lw, lb)


def kernel(x, edge_index, edge_attr, c1s1_W, c1s1_b, c1s2_W, c1s2_b, c1s3_W,
           c1s3_b, c2s1_W, c2s1_b, c2s2_W, c2s2_b, c2s3_W, c2s3_b,
           bn1_g, bn1_b, bn2_g, bn2_b, lin_W, lin_b):
    n, d = x.shape
    e = edge_index.shape[1]
    ept = -(-e // (NW * 2 * CH)) * 2 * CH
    e_pad = ept * NW
    nch_tot = e_pad // CH
    n_pad = NS * 640

    n0, n1 = _SPLIT
    assert (n0 + n1) * NS == nch_tot
    nmax = max(n0, n1)
    xtr = nmax * CH
    src2 = jnp.pad(edge_index[0], (0, e_pad + xtr - e)).reshape(-1, CH)
    dst2 = jnp.pad(edge_index[1], (0, e_pad + xtr - e)).reshape(-1, CH)
    ew2 = jnp.pad(edge_attr, (0, e_pad + xtr - e)).reshape(-1, CH)

    norm2 = _sc_pre(src2, dst2, ew2, n_pad, nch_tot)

    p1p = _sc_prop(x, src2, dst2, norm2, n0, n1)
    p2p = _sc_prop(p1p, src2, dst2, norm2, n0, n1, ((0, n_pad),))

    z128 = jnp.zeros((128, 400), jnp.float32)
    w384 = jnp.concatenate([
        jnp.concatenate([c1s1_W[0], c1s2_W[0], c1s3_W[0] - c1s3_W[2]], axis=1),
        jnp.concatenate([z128, c1s2_W[1], c1s3_W[1]], axis=1),
        jnp.concatenate([z128, z128, 2.0 * c1s3_W[2]], axis=1)], axis=0)
    bias1 = jnp.concatenate([c1s1_b, c1s2_b, c1s3_b])
    pre, stats1 = _tc_l1pre(x, p1p[:n], p1p[n_pad:n_pad + n],
                            p2p[:n], p2p[n_pad:n_pad + n], w384, bias1)

    def pad128(w):
        return jnp.pad(w, ((0, 0), (0, 28)))
    whb = jnp.concatenate([c2s1_W[0], c2s2_W[0], c2s3_W[0] - c2s3_W[2]], axis=1)
    bias2 = jnp.concatenate([c2s1_b, c2s2_b, c2s3_b])
    u1, u2, u3, hb = _tc_l1post(pre, stats1, bn1_g, bn1_b,
                                pad128(c2s2_W[1]), pad128(c2s3_W[1]),
                                pad128(c2s3_W[2]), whb, bias2)

    u = jnp.concatenate([u1, u2, u3], axis=0)
    up = _sc_prop(u, src2, dst2, norm2, n0, n1, ((0,), (n,), (2 * n,)))
    zp = _sc_prop(up, src2, dst2, norm2, n0, n1, ((4 * n_pad, 5 * n_pad),))

    out2, stats2 = _tc_fpre(hb, up[:n], up[n_pad:n_pad + n],
                            up[2 * n_pad:2 * n_pad + n],
                            up[3 * n_pad:3 * n_pad + n],
                            zp[:n], zp[n_pad:n_pad + n])
    return _tc_fpost(out2, stats2, bn2_g, bn2_b, lin_W, lin_b)


# speedup vs baseline: 1.0563x; 1.0563x over previous
import functools
import jax
import jax.numpy as jnp
from jax import lax
from jax.experimental import pallas as pl
from jax.experimental.pallas import tpu as pltpu
from jax.experimental.pallas import tpu_sc as plsc

NC = 2
NS = 16
NW = NC * NS
CH = 128
_SPLIT = (56, 24)


def _rsqrt16(d):
    ib = plsc.bitcast(d, jnp.int32)
    y = plsc.bitcast(jnp.int32(0x5F3759DF) - (ib >> 1), jnp.float32)
    for _ in range(4):
        y = y * (1.5 - 0.5 * d * y * y)
    return jnp.where(d > 0, y, 0.0)


def _sc_pre_body(n_pad, ept, src2_hbm, dst2_hbm, ew2_hbm, norm_hbm,
                 deg_sh, dis_sh, zbuf, zbuf2, dis_v, srcd_v, ewd_v,
                 srcn_v, dstn_v, ewn_v, nrm_v, sem):
    cid = lax.axis_index("c")
    sid = lax.axis_index("s")
    wid = cid * NS + sid
    rows_pt = n_pad // NS
    nch = ept // CH
    nchd = nch * NC

    for j in range(rows_pt // 16):
        zbuf[pl.ds(j * 16, 16)] = jnp.zeros((16,), jnp.float32)
    pltpu.sync_copy(zbuf, deg_sh.at[pl.ds(sid * rows_pt, rows_pt)])
    plsc.subcore_barrier()

    pltpu.sync_copy(src2_hbm.at[pl.ds(sid * nchd, nchd)], srcd_v)
    pltpu.sync_copy(ew2_hbm.at[pl.ds(sid * nchd, nchd)], ewd_v)

    def fire(k, _):
        pltpu.async_copy(ewd_v.at[k], deg_sh.at[srcd_v.at[k]], sem, add=True)
        return 0

    def drain(k, _):
        pltpu.make_async_copy(ewd_v.at[0], deg_sh.at[srcd_v.at[0]], sem).wait()
        return 0

    lax.fori_loop(0, nchd, fire, 0)
    lax.fori_loop(0, nchd, drain, 0)
    plsc.subcore_barrier()

    pltpu.sync_copy(deg_sh.at[pl.ds(sid * rows_pt, rows_pt)], zbuf)
    for j in range(rows_pt // 16):
        zbuf2[pl.ds(j * 16, 16)] = _rsqrt16(zbuf[pl.ds(j * 16, 16)])
    pltpu.sync_copy(zbuf2, dis_sh.at[pl.ds(sid * rows_pt, rows_pt)])
    plsc.subcore_barrier()
    pltpu.sync_copy(dis_sh, dis_v)

    pltpu.sync_copy(src2_hbm.at[pl.ds(wid * nch, nch)], srcn_v)
    pltpu.sync_copy(dst2_hbm.at[pl.ds(wid * nch, nch)], dstn_v)
    pltpu.sync_copy(ew2_hbm.at[pl.ds(wid * nch, nch)], ewn_v)

    def nchunk(k, _):
        for j in range(CH // 16):
            sl = pl.ds(j * 16, 16)
            a = plsc.load_gather(dis_v, [srcn_v[k, sl]])
            b = plsc.load_gather(dis_v, [dstn_v[k, sl]])
            nrm_v[k, sl] = -(a * ewn_v[k, sl]) * b
        return 0

    lax.fori_loop(0, nch, nchunk, 0)
    pltpu.sync_copy(nrm_v, norm_hbm.at[pl.ds(wid * nch, nch)])


def _sc_pre(src2, dst2, ew2, n_pad, nch_tot):
    nrows = src2.shape[0]
    nch = nch_tot // NW
    ept = nch * CH
    mesh = plsc.VectorSubcoreMesh(core_axis_name="c", subcore_axis_name="s")
    kfn = pl.kernel(
        functools.partial(_sc_pre_body, n_pad, ept),
        out_type=jax.ShapeDtypeStruct((nrows, CH), jnp.float32),
        mesh=mesh,
        compiler_params=pltpu.CompilerParams(needs_layout_passes=False),
        scratch_types=[
            pltpu.VMEM_SHARED((n_pad,), jnp.float32),
            pltpu.VMEM_SHARED((n_pad,), jnp.float32),
            pltpu.VMEM((n_pad // NS,), jnp.float32),
            pltpu.VMEM((n_pad // NS,), jnp.float32),
            pltpu.VMEM((n_pad,), jnp.float32),
            pltpu.VMEM((nch * NC, CH), jnp.int32),
            pltpu.VMEM((nch * NC, CH), jnp.float32),
            pltpu.VMEM((nch, CH), jnp.int32),
            pltpu.VMEM((nch, CH), jnp.int32),
            pltpu.VMEM((nch, CH), jnp.float32),
            pltpu.VMEM((nch, CH), jnp.float32),
            pltpu.SemaphoreType.DMA,
        ],
    )
    return kfn(src2, dst2, ew2)


def _sc_prop_body(n_pad, dc, n0, n1, bases, y_hbm, src2_hbm, dst2_hbm,
                  nrm2_hbm, out_hbm, acc_sh, rows_a, rows_b, src2_v, dst2_v,
                  nrm_v, shi_a, shi_a2, shi_b, shi_b2,
                  sem_a, sem_a2, sem_b, sem_b2, sem_sa, sem_sb):
    cid = lax.axis_index("c")
    sid = lax.axis_index("s")
    rows_pt = n_pad // NS
    row0 = jnp.where(cid == 0, sid * n0, NS * n0 + sid * n1)
    nsc_self = jnp.where(cid == 0, n0 // 8, n1 // 8)

    def zrow(e, _):
        for j in range(dc // 16):
            rows_a[e, pl.ds(j * 16, 16)] = jnp.zeros((16,), jnp.float32)
        return 0

    for p, gb in enumerate(bases):
        dual = len(gb) == 2

        def issue(k, base, rows, shi, sem, add=False):
            if base == 0:
                pltpu.async_copy(y_hbm.at[src2_v.at[k]], rows, sem, add=add)
            else:
                for j in range(CH // 16):
                    sl = pl.ds(j * 16, 16)
                    shi[sl] = src2_v[k, sl] + base
                pltpu.async_copy(y_hbm.at[shi], rows, sem, add=add)

        def wait(rows, sem):
            pltpu.make_async_copy(y_hbm.at[src2_v.at[0]], rows, sem).wait()

        def scale_scatter(k, rows, ssem):
            @plsc.parallel_loop(0, CH, unroll=4)
            def scale(e):
                nv = plsc.load_gather(
                    nrm_v, [jnp.zeros((16,), jnp.int32) + k,
                            jnp.zeros((16,), jnp.int32) + e])
                for j in range(dc // 16):
                    sl = pl.ds(j * 16, 16)
                    rows[e, sl] = rows[e, sl] * nv

            pltpu.async_copy(rows, acc_sh.at[dst2_v.at[k]], ssem, add=True)

        def swait(rows, ssem):
            pltpu.make_async_copy(rows, acc_sh.at[dst2_v.at[0]], ssem).wait()

        lax.fori_loop(0, CH, zrow, 0)
        off = 0
        while off < rows_pt:
            sz = min(CH, rows_pt - off)
            pltpu.sync_copy(rows_a.at[pl.ds(0, sz)],
                            acc_sh.at[pl.ds(sid * rows_pt + off, sz)])
            off += sz
        plsc.subcore_barrier()

        def super_chunk(q, _):
            srow = row0 + q * 8
            pltpu.sync_copy(src2_hbm.at[pl.ds(srow, 8)], src2_v)
            pltpu.sync_copy(dst2_hbm.at[pl.ds(srow, 8)], dst2_v)
            pltpu.sync_copy(nrm2_hbm.at[pl.ds(srow, 8)], nrm_v)
            issue(0, gb[0], rows_a, shi_a, sem_a)
            for gg in range(4):
                k0 = 2 * gg
                wait(rows_a, sem_a)
                if dual:
                    issue(k0, gb[1], rows_a, shi_a2, sem_a2, add=True)
                if gg > 0:
                    swait(rows_b, sem_sb)
                issue(k0 + 1, gb[0], rows_b, shi_b, sem_b)
                if dual:
                    wait(rows_a, sem_a2)
                scale_scatter(k0, rows_a, sem_sa)
                wait(rows_b, sem_b)
                if dual:
                    issue(k0 + 1, gb[1], rows_b, shi_b2, sem_b2, add=True)
                if gg < 3:
                    swait(rows_a, sem_sa)
                    issue(k0 + 2, gb[0], rows_a, shi_a, sem_a)
                if dual:
                    wait(rows_b, sem_b2)
                scale_scatter(k0 + 1, rows_b, sem_sb)
            swait(rows_a, sem_sa)
            swait(rows_b, sem_sb)
            return 0

        lax.fori_loop(0, nsc_self, super_chunk, 0)
        plsc.subcore_barrier()

        orow = (p * NC + cid) * n_pad + sid * rows_pt
        off = 0
        while off < rows_pt:
            sz = min(CH, rows_pt - off)
            pltpu.sync_copy(acc_sh.at[pl.ds(sid * rows_pt + off, sz)],
                            rows_a.at[pl.ds(0, sz)])
            pltpu.sync_copy(rows_a.at[pl.ds(0, sz)],
                            out_hbm.at[pl.ds(orow + off, sz)])
            off += sz


def _sc_prop(y, src2, dst2, norm2, n0, n1, bases=((0,),)):
    n_rows_y, dc = y.shape
    npass = len(bases)
    n_pad = NS * 640
    nmax = max(n0, n1)
    assert n0 % 8 == 0 and n1 % 8 == 0
    mesh = plsc.VectorSubcoreMesh(core_axis_name="c", subcore_axis_name="s")
    kfn = pl.kernel(
        functools.partial(_sc_prop_body, n_pad, dc, n0, n1, bases),
        out_type=jax.ShapeDtypeStruct((npass * NC * n_pad, dc), jnp.float32),
        mesh=mesh,
        compiler_params=pltpu.CompilerParams(needs_layout_passes=False),
        scratch_types=[
            pltpu.VMEM_SHARED((n_pad, dc), jnp.float32),
            pltpu.VMEM((CH, dc), jnp.float32),
            pltpu.VMEM((CH, dc), jnp.float32),
            pltpu.VMEM((8, CH), jnp.int32),
            pltpu.VMEM((8, CH), jnp.int32),
            pltpu.VMEM((8, CH), jnp.float32),
            pltpu.VMEM((CH,), jnp.int32),
            pltpu.VMEM((CH,), jnp.int32),
            pltpu.VMEM((CH,), jnp.int32),
            pltpu.VMEM((CH,), jnp.int32),
            pltpu.SemaphoreType.DMA,
            pltpu.SemaphoreType.DMA,
            pltpu.SemaphoreType.DMA,
            pltpu.SemaphoreType.DMA,
            pltpu.SemaphoreType.DMA,
            pltpu.SemaphoreType.DMA,
        ],
    )
    return kfn(y, src2, dst2, norm2)


def _l1pre_body(x_ref, p1a_ref, p1b_ref, p2a_ref, p2b_ref, w_ref, b_ref,
                pre_ref, st_ref):
    xc = jnp.concatenate([x_ref[...], p1a_ref[...] + p1b_ref[...],
                          p2a_ref[...] + p2b_ref[...]], axis=1)
    pre = jnp.dot(xc, w_ref[...], preferred_element_type=jnp.float32) + b_ref[...]
    pre_ref[...] = pre

    @pl.when(pl.program_id(0) == 0)
    def _():
        st_ref[...] = jnp.zeros_like(st_ref)

    st_ref[...] += jnp.stack([jnp.sum(pre, axis=0), jnp.sum(pre * pre, axis=0)])


def _tc_l1pre(x, p1a, p1b, p2a, p2b, w384, bias1, bm=1000):
    n = x.shape[0]
    f = w384.shape[1]
    return pl.pallas_call(
        _l1pre_body,
        out_shape=[jax.ShapeDtypeStruct((n, f), jnp.float32),
                   jax.ShapeDtypeStruct((2, f), jnp.float32)],
        grid=(n // bm,),
        in_specs=[pl.BlockSpec((bm, 128), lambda i: (i, 0)),
                  pl.BlockSpec((bm, 128), lambda i: (i, 0)),
                  pl.BlockSpec((bm, 128), lambda i: (i, 0)),
                  pl.BlockSpec((bm, 128), lambda i: (i, 0)),
                  pl.BlockSpec((bm, 128), lambda i: (i, 0)),
                  pl.BlockSpec((384, f), lambda i: (0, 0)),
                  pl.BlockSpec((f,), lambda i: (0,))],
        out_specs=[pl.BlockSpec((bm, f), lambda i: (i, 0)),
                   pl.BlockSpec((2, f), lambda i: (0, 0))],
    )(x, p1a, p1b, p2a, p2b, w384, bias1)


def _l1post_body(n_rows, pre_ref, st_ref, g_ref, bb_ref, wu1_ref, wu2_ref,
                 wu3_ref, whb_ref, b2_ref, u1_ref, u2_ref, u3_ref, hb_ref):
    m = st_ref[0, :] / n_rows
    var = st_ref[1, :] / n_rows - m * m
    scale = g_ref[...] * lax.rsqrt(var + 1e-5)
    shift = bb_ref[...] - m * scale
    h = jnp.maximum(pre_ref[...] * scale + shift, 0.0)
    u1_ref[...] = jnp.dot(h, wu1_ref[...], preferred_element_type=jnp.float32)
    u2_ref[...] = jnp.dot(h, wu2_ref[...], preferred_element_type=jnp.float32)
    u3_ref[...] = jnp.dot(h, wu3_ref[...], preferred_element_type=jnp.float32)
    hb_ref[...] = jnp.dot(h, whb_ref[...], preferred_element_type=jnp.float32) + b2_ref[...]


def _tc_l1post(pre, stats, g, b, wu1, wu2, wu3, whb, bias2, bm=1000):
    n, f = pre.shape
    du = wu1.shape[1]
    return pl.pallas_call(
        functools.partial(_l1post_body, float(n)),
        out_shape=[jax.ShapeDtypeStruct((n, du), jnp.float32),
                   jax.ShapeDtypeStruct((n, du), jnp.float32),
                   jax.ShapeDtypeStruct((n, du), jnp.float32),
                   jax.ShapeDtypeStruct((n, 300), jnp.float32)],
        grid=(n // bm,),
        in_specs=[pl.BlockSpec((bm, f), lambda i: (i, 0)),
                  pl.BlockSpec((2, f), lambda i: (0, 0)),
                  pl.BlockSpec((f,), lambda i: (0,)),
                  pl.BlockSpec((f,), lambda i: (0,)),
                  pl.BlockSpec((f, du), lambda i: (0, 0)),
                  pl.BlockSpec((f, du), lambda i: (0, 0)),
                  pl.BlockSpec((f, du), lambda i: (0, 0)),
                  pl.BlockSpec((f, 300), lambda i: (0, 0)),
                  pl.BlockSpec((300,), lambda i: (0,))],
        out_specs=[pl.BlockSpec((bm, du), lambda i: (i, 0)),
                   pl.BlockSpec((bm, du), lambda i: (i, 0)),
                   pl.BlockSpec((bm, du), lambda i: (i, 0)),
                   pl.BlockSpec((bm, 300), lambda i: (i, 0))],
    )(pre, stats, g, b, wu1, wu2, wu3, whb, bias2)


def _fpre_body(hb_ref, v1a_ref, v1b_ref, v2a_ref, v2b_ref, za_ref, zb_ref,
               o_ref, st_ref):
    bm = hb_ref.shape[0]
    v1 = (v1a_ref[...] + v1b_ref[...])[:, :100]
    v23 = (v2a_ref[...] + v2b_ref[...] + 2.0 * (za_ref[...] + zb_ref[...]))[:, :100]
    add = jnp.concatenate([jnp.zeros((bm, 100), jnp.float32), v1, v23], axis=1)
    o = hb_ref[...] + add
    o_ref[...] = o

    @pl.when(pl.program_id(0) == 0)
    def _():
        st_ref[...] = jnp.zeros_like(st_ref)

    st_ref[...] += jnp.stack([jnp.sum(o, axis=0), jnp.sum(o * o, axis=0)])


def _tc_fpre(hb, v1a, v1b, v2a, v2b, za, zb, bm=1000):
    n = hb.shape[0]
    du = v1a.shape[1]
    return pl.pallas_call(
        _fpre_body,
        out_shape=[jax.ShapeDtypeStruct((n, 300), jnp.float32),
                   jax.ShapeDtypeStruct((2, 300), jnp.float32)],
        grid=(n // bm,),
        in_specs=[pl.BlockSpec((bm, 300), lambda i: (i, 0))] +
                 [pl.BlockSpec((bm, du), lambda i: (i, 0))] * 6,
        out_specs=[pl.BlockSpec((bm, 300), lambda i: (i, 0)),
                   pl.BlockSpec((2, 300), lambda i: (0, 0))],
    )(hb, v1a, v1b, v2a, v2b, za, zb)


def _fpost_body(n_rows, o2_ref, st_ref, g_ref, bb_ref, lw_ref, lb_ref, o_ref):
    m = st_ref[0, :] / n_rows
    var = st_ref[1, :] / n_rows - m * m
    scale = g_ref[...] * lax.rsqrt(var + 1e-5)
    shift = bb_ref[...] - m * scale
    h2 = o2_ref[...] * scale + shift
    logits = jnp.dot(h2, lw_ref[...], preferred_element_type=jnp.float32) + lb_ref[...]
    mx = jnp.max(logits, axis=1, keepdims=True)
    lse = jnp.log(jnp.sum(jnp.exp(logits - mx), axis=1, keepdims=True)) + mx
    o_ref[...] = logits - lse


def _tc_fpost(out2, stats, g, b, lw, lb, bm=1000):
    n = out2.shape[0]
    k = lw.shape[1]
    return pl.pallas_call(
        functools.partial(_fpost_body, float(n)),
        out_shape=jax.ShapeDtypeStruct((n, k), jnp.float32),
        grid=(n // bm,),
        in_specs=[pl.BlockSpec((bm, 300), lambda i: (i, 0)),
                  pl.BlockSpec((2, 300), lambda i: (0, 0)),
                  pl.BlockSpec((300,), lambda i: (0,)),
                  pl.BlockSpec((300,), lambda i: (0,)),
                  pl.BlockSpec((300, k), lambda i: (0, 0)),
                  pl.BlockSpec((k,), lambda i: (0,))],
        out_specs=pl.BlockSpec((bm, k), lambda i: (i, 0)),
    )(out2, stats, g, b, lw, lb)


def kernel(x, edge_index, edge_attr, c1s1_W, c1s1_b, c1s2_W, c1s2_b, c1s3_W,
           c1s3_b, c2s1_W, c2s1_b, c2s2_W, c2s2_b, c2s3_W, c2s3_b,
           bn1_g, bn1_b, bn2_g, bn2_b, lin_W, lin_b):
    n, d = x.shape
    e = edge_index.shape[1]
    ept = -(-e // (NW * 2 * CH)) * 2 * CH
    e_pad = ept * NW
    nch_tot = e_pad // CH
    n_pad = NS * 640

    n0, n1 = _SPLIT
    assert (n0 + n1) * NS == nch_tot
    nmax = max(n0, n1)
    xtr = nmax * CH
    src2 = jnp.pad(edge_index[0], (0, e_pad + xtr - e)).reshape(-1, CH)
    dst2 = jnp.pad(edge_index[1], (0, e_pad + xtr - e)).reshape(-1, CH)
    ew2 = jnp.pad(edge_attr, (0, e_pad + xtr - e)).reshape(-1, CH)

    norm2 = _sc_pre(src2, dst2, ew2, n_pad, nch_tot)

    p1p = _sc_prop(x, src2, dst2, norm2, n0, n1)
    p2p = _sc_prop(p1p, src2, dst2, norm2, n0, n1, ((0, n_pad),))

    z128 = jnp.zeros((128, 400), jnp.float32)
    w384 = jnp.concatenate([
        jnp.concatenate([c1s1_W[0], c1s2_W[0], c1s3_W[0] - c1s3_W[2]], axis=1),
        jnp.concatenate([z128, c1s2_W[1], c1s3_W[1]], axis=1),
        jnp.concatenate([z128, z128, 2.0 * c1s3_W[2]], axis=1)], axis=0)
    bias1 = jnp.concatenate([c1s1_b, c1s2_b, c1s3_b])
    pre, stats1 = _tc_l1pre(x, p1p[:n], p1p[n_pad:n_pad + n],
                            p2p[:n], p2p[n_pad:n_pad + n], w384, bias1)

    def pad128(w):
        return jnp.pad(w, ((0, 0), (0, 28)))
    whb = jnp.concatenate([c2s1_W[0], c2s2_W[0], c2s3_W[0] - c2s3_W[2]], axis=1)
    bias2 = jnp.concatenate([c2s1_b, c2s2_b, c2s3_b])
    u1, u2, u3, hb = _tc_l1post(pre, stats1, bn1_g, bn1_b,
                                pad128(c2s2_W[1]), pad128(c2s3_W[1]),
                                pad128(c2s3_W[2]), whb, bias2)

    u = jnp.concatenate([u1, u2, u3], axis=0)
    up = _sc_prop(u, src2, dst2, norm2, n0, n1, ((0,), (n,), (2 * n,)))
    zp = _sc_prop(up, src2, dst2, norm2, n0, n1, ((4 * n_pad, 5 * n_pad),))

    out2, stats2 = _tc_fpre(hb, up[:n], up[n_pad:n_pad + n],
                            up[2 * n_pad:2 * n_pad + n],
                            up[3 * n_pad:3 * n_pad + n],
                            zp[:n], zp[n_pad:n_pad + n])
    return _tc_fpost(out2, stats2, bn2_g, bn2_b, lin_W, lin_b)


# speedup vs baseline: 1.0715x; 1.0144x over previous
import functools
import jax
import jax.numpy as jnp
from jax import lax
from jax.experimental import pallas as pl
from jax.experimental.pallas import tpu as pltpu
from jax.experimental.pallas import tpu_sc as plsc

NC = 2
NS = 16
NW = NC * NS
CH = 128
_SPLIT = (56, 24)


def _rsqrt16(d):
    ib = plsc.bitcast(d, jnp.int32)
    y = plsc.bitcast(jnp.int32(0x5F3759DF) - (ib >> 1), jnp.float32)
    for _ in range(4):
        y = y * (1.5 - 0.5 * d * y * y)
    return jnp.where(d > 0, y, 0.0)


def _sc_pre_body(n_pad, ept, src2_hbm, dst2_hbm, ew2_hbm, norm_hbm,
                 deg_sh, dis_sh, zbuf, zbuf2, dis_v, srcd_v, ewd_v,
                 srcn_v, dstn_v, ewn_v, nrm_v, sem):
    cid = lax.axis_index("c")
    sid = lax.axis_index("s")
    wid = cid * NS + sid
    rows_pt = n_pad // NS
    nch = ept // CH
    nchd = nch * NC

    for j in range(rows_pt // 16):
        zbuf[pl.ds(j * 16, 16)] = jnp.zeros((16,), jnp.float32)
    pltpu.sync_copy(zbuf, deg_sh.at[pl.ds(sid * rows_pt, rows_pt)])
    plsc.subcore_barrier()

    pltpu.sync_copy(src2_hbm.at[pl.ds(sid * nchd, nchd)], srcd_v)
    pltpu.sync_copy(ew2_hbm.at[pl.ds(sid * nchd, nchd)], ewd_v)

    def fire(k, _):
        pltpu.async_copy(ewd_v.at[k], deg_sh.at[srcd_v.at[k]], sem, add=True)
        return 0

    def drain(k, _):
        pltpu.make_async_copy(ewd_v.at[0], deg_sh.at[srcd_v.at[0]], sem).wait()
        return 0

    lax.fori_loop(0, nchd, fire, 0)
    lax.fori_loop(0, nchd, drain, 0)
    plsc.subcore_barrier()

    pltpu.sync_copy(deg_sh.at[pl.ds(sid * rows_pt, rows_pt)], zbuf)
    for j in range(rows_pt // 16):
        zbuf2[pl.ds(j * 16, 16)] = _rsqrt16(zbuf[pl.ds(j * 16, 16)])
    pltpu.sync_copy(zbuf2, dis_sh.at[pl.ds(sid * rows_pt, rows_pt)])
    plsc.subcore_barrier()
    pltpu.sync_copy(dis_sh, dis_v)

    pltpu.sync_copy(src2_hbm.at[pl.ds(wid * nch, nch)], srcn_v)
    pltpu.sync_copy(dst2_hbm.at[pl.ds(wid * nch, nch)], dstn_v)
    pltpu.sync_copy(ew2_hbm.at[pl.ds(wid * nch, nch)], ewn_v)

    def nchunk(k, _):
        for j in range(CH // 16):
            sl = pl.ds(j * 16, 16)
            a = plsc.load_gather(dis_v, [srcn_v[k, sl]])
            b = plsc.load_gather(dis_v, [dstn_v[k, sl]])
            nrm_v[k, sl] = -(a * ewn_v[k, sl]) * b
        return 0

    lax.fori_loop(0, nch, nchunk, 0)
    pltpu.sync_copy(nrm_v, norm_hbm.at[pl.ds(wid * nch, nch)])


def _sc_pre(src2, dst2, ew2, n_pad, nch_tot):
    nrows = src2.shape[0]
    nch = nch_tot // NW
    ept = nch * CH
    mesh = plsc.VectorSubcoreMesh(core_axis_name="c", subcore_axis_name="s")
    kfn = pl.kernel(
        functools.partial(_sc_pre_body, n_pad, ept),
        out_type=jax.ShapeDtypeStruct((nrows, CH), jnp.float32),
        mesh=mesh,
        compiler_params=pltpu.CompilerParams(needs_layout_passes=False),
        scratch_types=[
            pltpu.VMEM_SHARED((n_pad,), jnp.float32),
            pltpu.VMEM_SHARED((n_pad,), jnp.float32),
            pltpu.VMEM((n_pad // NS,), jnp.float32),
            pltpu.VMEM((n_pad // NS,), jnp.float32),
            pltpu.VMEM((n_pad,), jnp.float32),
            pltpu.VMEM((nch * NC, CH), jnp.int32),
            pltpu.VMEM((nch * NC, CH), jnp.float32),
            pltpu.VMEM((nch, CH), jnp.int32),
            pltpu.VMEM((nch, CH), jnp.int32),
            pltpu.VMEM((nch, CH), jnp.float32),
            pltpu.VMEM((nch, CH), jnp.float32),
            pltpu.SemaphoreType.DMA,
        ],
    )
    return kfn(src2, dst2, ew2)


def _sc_prop_body(n_pad, dc, n0, n1, bases, y_hbm, src2_hbm, dst2_hbm,
                  nrm2_hbm, out_hbm, acc_sh, rows_a, rows_b, src2_v, dst2_v,
                  nrm_v, shi_a, shi_a2, shi_b, shi_b2,
                  sem_a, sem_a2, sem_b, sem_b2):
    cid = lax.axis_index("c")
    sid = lax.axis_index("s")
    rows_pt = n_pad // NS
    row0 = jnp.where(cid == 0, sid * n0, NS * n0 + sid * n1)
    nsc_self = jnp.where(cid == 0, n0 // 8, n1 // 8)

    def zrow(e, _):
        for j in range(dc // 16):
            rows_a[e, pl.ds(j * 16, 16)] = jnp.zeros((16,), jnp.float32)
        return 0

    for p, gb in enumerate(bases):
        dual = len(gb) == 2

        def issue(k, base, rows, shi, sem, add=False):
            if base == 0:
                pltpu.async_copy(y_hbm.at[src2_v.at[k]], rows, sem, add=add)
            else:
                for j in range(CH // 16):
                    sl = pl.ds(j * 16, 16)
                    shi[sl] = src2_v[k, sl] + base
                pltpu.async_copy(y_hbm.at[shi], rows, sem, add=add)

        def wait(rows, sem):
            pltpu.make_async_copy(y_hbm.at[src2_v.at[0]], rows, sem).wait()

        def scale_scatter(k, rows):
            @plsc.parallel_loop(0, CH, unroll=4)
            def scale(e):
                nv = plsc.load_gather(
                    nrm_v, [jnp.zeros((16,), jnp.int32) + k,
                            jnp.zeros((16,), jnp.int32) + e])
                for j in range(dc // 16):
                    sl = pl.ds(j * 16, 16)
                    rows[e, sl] = rows[e, sl] * nv

            pltpu.sync_copy(rows, acc_sh.at[dst2_v.at[k]], add=True)

        lax.fori_loop(0, CH, zrow, 0)
        off = 0
        while off < rows_pt:
            sz = min(CH, rows_pt - off)
            pltpu.sync_copy(rows_a.at[pl.ds(0, sz)],
                            acc_sh.at[pl.ds(sid * rows_pt + off, sz)])
            off += sz
        plsc.subcore_barrier()

        def super_chunk(q, _):
            srow = row0 + q * 8
            pltpu.sync_copy(src2_hbm.at[pl.ds(srow, 8)], src2_v)
            pltpu.sync_copy(dst2_hbm.at[pl.ds(srow, 8)], dst2_v)
            pltpu.sync_copy(nrm2_hbm.at[pl.ds(srow, 8)], nrm_v)
            issue(0, gb[0], rows_a, shi_a, sem_a)
            for gg in range(4):
                k0 = 2 * gg
                wait(rows_a, sem_a)
                if dual:
                    issue(k0, gb[1], rows_a, shi_a2, sem_a2, add=True)
                issue(k0 + 1, gb[0], rows_b, shi_b, sem_b)
                if dual:
                    wait(rows_a, sem_a2)
                scale_scatter(k0, rows_a)
                wait(rows_b, sem_b)
                if dual:
                    issue(k0 + 1, gb[1], rows_b, shi_b2, sem_b2, add=True)
                if gg < 3:
                    issue(k0 + 2, gb[0], rows_a, shi_a, sem_a)
                if dual:
                    wait(rows_b, sem_b2)
                scale_scatter(k0 + 1, rows_b)
            return 0

        lax.fori_loop(0, nsc_self, super_chunk, 0)
        plsc.subcore_barrier()

        orow = (p * NC + cid) * n_pad + sid * rows_pt
        off = 0
        while off < rows_pt:
            sz = min(CH, rows_pt - off)
            pltpu.sync_copy(acc_sh.at[pl.ds(sid * rows_pt + off, sz)],
                            rows_a.at[pl.ds(0, sz)])
            pltpu.sync_copy(rows_a.at[pl.ds(0, sz)],
                            out_hbm.at[pl.ds(orow + off, sz)])
            off += sz


def _sc_prop(y, src2, dst2, norm2, n0, n1, bases=((0,),)):
    n_rows_y, dc = y.shape
    npass = len(bases)
    n_pad = NS * 640
    nmax = max(n0, n1)
    assert n0 % 8 == 0 and n1 % 8 == 0
    mesh = plsc.VectorSubcoreMesh(core_axis_name="c", subcore_axis_name="s")
    kfn = pl.kernel(
        functools.partial(_sc_prop_body, n_pad, dc, n0, n1, bases),
        out_type=jax.ShapeDtypeStruct((npass * NC * n_pad, dc), jnp.float32),
        mesh=mesh,
        compiler_params=pltpu.CompilerParams(needs_layout_passes=False),
        scratch_types=[
            pltpu.VMEM_SHARED((n_pad, dc), jnp.float32),
            pltpu.VMEM((CH, dc), jnp.float32),
            pltpu.VMEM((CH, dc), jnp.float32),
            pltpu.VMEM((8, CH), jnp.int32),
            pltpu.VMEM((8, CH), jnp.int32),
            pltpu.VMEM((8, CH), jnp.float32),
            pltpu.VMEM((CH,), jnp.int32),
            pltpu.VMEM((CH,), jnp.int32),
            pltpu.VMEM((CH,), jnp.int32),
            pltpu.VMEM((CH,), jnp.int32),
            pltpu.SemaphoreType.DMA,
            pltpu.SemaphoreType.DMA,
            pltpu.SemaphoreType.DMA,
            pltpu.SemaphoreType.DMA,
        ],
    )
    return kfn(y, src2, dst2, norm2)


def _l1pre_body(x_ref, p1a_ref, p1b_ref, p2a_ref, p2b_ref, w_ref, b_ref,
                pre_ref, st_ref):
    xc = jnp.concatenate([x_ref[...], p1a_ref[...] + p1b_ref[...],
                          p2a_ref[...] + p2b_ref[...]], axis=1)
    pre = jnp.dot(xc, w_ref[...], preferred_element_type=jnp.float32) + b_ref[...]
    pre_ref[...] = pre

    @pl.when(pl.program_id(0) == 0)
    def _():
        st_ref[...] = jnp.zeros_like(st_ref)

    st_ref[...] += jnp.stack([jnp.sum(pre, axis=0), jnp.sum(pre * pre, axis=0)])


def _tc_l1pre(x, p1a, p1b, p2a, p2b, w384, bias1, bm=1000):
    n = x.shape[0]
    f = w384.shape[1]
    return pl.pallas_call(
        _l1pre_body,
        out_shape=[jax.ShapeDtypeStruct((n, f), jnp.float32),
                   jax.ShapeDtypeStruct((2, f), jnp.float32)],
        grid=(n // bm,),
        in_specs=[pl.BlockSpec((bm, 128), lambda i: (i, 0)),
                  pl.BlockSpec((bm, 128), lambda i: (i, 0)),
                  pl.BlockSpec((bm, 128), lambda i: (i, 0)),
                  pl.BlockSpec((bm, 128), lambda i: (i, 0)),
                  pl.BlockSpec((bm, 128), lambda i: (i, 0)),
                  pl.BlockSpec((384, f), lambda i: (0, 0)),
                  pl.BlockSpec((f,), lambda i: (0,))],
        out_specs=[pl.BlockSpec((bm, f), lambda i: (i, 0)),
                   pl.BlockSpec((2, f), lambda i: (0, 0))],
    )(x, p1a, p1b, p2a, p2b, w384, bias1)


def _l1post_body(n_rows, pre_ref, st_ref, g_ref, bb_ref, wu1_ref, wu2_ref,
                 wu3_ref, whb_ref, b2_ref, u1_ref, u2_ref, u3_ref, hb_ref):
    m = st_ref[0, :] / n_rows
    var = st_ref[1, :] / n_rows - m * m
    scale = g_ref[...] * lax.rsqrt(var + 1e-5)
    shift = bb_ref[...] - m * scale
    h = jnp.maximum(pre_ref[...] * scale + shift, 0.0)
    u1_ref[...] = jnp.dot(h, wu1_ref[...], preferred_element_type=jnp.float32)
    u2_ref[...] = jnp.dot(h, wu2_ref[...], preferred_element_type=jnp.float32)
    u3_ref[...] = jnp.dot(h, wu3_ref[...], preferred_element_type=jnp.float32)
    hb_ref[...] = jnp.dot(h, whb_ref[...], preferred_element_type=jnp.float32) + b2_ref[...]


def _tc_l1post(pre, stats, g, b, wu1, wu2, wu3, whb, bias2, bm=1000):
    n, f = pre.shape
    du = wu1.shape[1]
    return pl.pallas_call(
        functools.partial(_l1post_body, float(n)),
        out_shape=[jax.ShapeDtypeStruct((n, du), jnp.float32),
                   jax.ShapeDtypeStruct((n, du), jnp.float32),
                   jax.ShapeDtypeStruct((n, du), jnp.float32),
                   jax.ShapeDtypeStruct((n, 300), jnp.float32)],
        grid=(n // bm,),
        in_specs=[pl.BlockSpec((bm, f), lambda i: (i, 0)),
                  pl.BlockSpec((2, f), lambda i: (0, 0)),
                  pl.BlockSpec((f,), lambda i: (0,)),
                  pl.BlockSpec((f,), lambda i: (0,)),
                  pl.BlockSpec((f, du), lambda i: (0, 0)),
                  pl.BlockSpec((f, du), lambda i: (0, 0)),
                  pl.BlockSpec((f, du), lambda i: (0, 0)),
                  pl.BlockSpec((f, 300), lambda i: (0, 0)),
                  pl.BlockSpec((300,), lambda i: (0,))],
        out_specs=[pl.BlockSpec((bm, du), lambda i: (i, 0)),
                   pl.BlockSpec((bm, du), lambda i: (i, 0)),
                   pl.BlockSpec((bm, du), lambda i: (i, 0)),
                   pl.BlockSpec((bm, 300), lambda i: (i, 0))],
    )(pre, stats, g, b, wu1, wu2, wu3, whb, bias2)


def _fpre_body(hb_ref, v1a_ref, v1b_ref, v2a_ref, v2b_ref, za_ref, zb_ref,
               o_ref, st_ref):
    bm = hb_ref.shape[0]
    v1 = (v1a_ref[...] + v1b_ref[...])[:, :100]
    v23 = (v2a_ref[...] + v2b_ref[...] + 2.0 * (za_ref[...] + zb_ref[...]))[:, :100]
    add = jnp.concatenate([jnp.zeros((bm, 100), jnp.float32), v1, v23], axis=1)
    o = hb_ref[...] + add
    o_ref[...] = o

    @pl.when(pl.program_id(0) == 0)
    def _():
        st_ref[...] = jnp.zeros_like(st_ref)

    st_ref[...] += jnp.stack([jnp.sum(o, axis=0), jnp.sum(o * o, axis=0)])


def _tc_fpre(hb, v1a, v1b, v2a, v2b, za, zb, bm=1000):
    n = hb.shape[0]
    du = v1a.shape[1]
    return pl.pallas_call(
        _fpre_body,
        out_shape=[jax.ShapeDtypeStruct((n, 300), jnp.float32),
                   jax.ShapeDtypeStruct((2, 300), jnp.float32)],
        grid=(n // bm,),
        in_specs=[pl.BlockSpec((bm, 300), lambda i: (i, 0))] +
                 [pl.BlockSpec((bm, du), lambda i: (i, 0))] * 6,
        out_specs=[pl.BlockSpec((bm, 300), lambda i: (i, 0)),
                   pl.BlockSpec((2, 300), lambda i: (0, 0))],
    )(hb, v1a, v1b, v2a, v2b, za, zb)


def _fpost_body(n_rows, o2_ref, st_ref, g_ref, bb_ref, lw_ref, lb_ref, o_ref):
    m = st_ref[0, :] / n_rows
    var = st_ref[1, :] / n_rows - m * m
    scale = g_ref[...] * lax.rsqrt(var + 1e-5)
    shift = bb_ref[...] - m * scale
    h2 = o2_ref[...] * scale + shift
    logits = jnp.dot(h2, lw_ref[...], preferred_element_type=jnp.float32) + lb_ref[...]
    mx = jnp.max(logits, axis=1, keepdims=True)
    lse = jnp.log(jnp.sum(jnp.exp(logits - mx), axis=1, keepdims=True)) + mx
    o_ref[...] = logits - lse


def _tc_fpost(out2, stats, g, b, lw, lb, bm=1000):
    n = out2.shape[0]
    k = lw.shape[1]
    return pl.pallas_call(
        functools.partial(_fpost_body, float(n)),
        out_shape=jax.ShapeDtypeStruct((n, k), jnp.float32),
        grid=(n // bm,),
        in_specs=[pl.BlockSpec((bm, 300), lambda i: (i, 0)),
                  pl.BlockSpec((2, 300), lambda i: (0, 0)),
                  pl.BlockSpec((300,), lambda i: (0,)),
                  pl.BlockSpec((300,), lambda i: (0,)),
                  pl.BlockSpec((300, k), lambda i: (0, 0)),
                  pl.BlockSpec((k,), lambda i: (0,))],
        out_specs=pl.BlockSpec((bm, k), lambda i: (i, 0)),
    )(out2, stats, g, b, lw, lb)


def kernel(x, edge_index, edge_attr, c1s1_W, c1s1_b, c1s2_W, c1s2_b, c1s3_W,
           c1s3_b, c2s1_W, c2s1_b, c2s2_W, c2s2_b, c2s3_W, c2s3_b,
           bn1_g, bn1_b, bn2_g, bn2_b, lin_W, lin_b):
    n, d = x.shape
    e = edge_index.shape[1]
    ept = -(-e // (NW * 2 * CH)) * 2 * CH
    e_pad = ept * NW
    nch_tot = e_pad // CH
    n_pad = NS * 640

    n0, n1 = _SPLIT
    assert (n0 + n1) * NS == nch_tot
    nmax = max(n0, n1)
    xtr = nmax * CH
    src2 = jnp.pad(edge_index[0], (0, e_pad + xtr - e)).reshape(-1, CH)
    dst2 = jnp.pad(edge_index[1], (0, e_pad + xtr - e)).reshape(-1, CH)
    ew2 = jnp.pad(edge_attr, (0, e_pad + xtr - e)).reshape(-1, CH)

    norm2 = _sc_pre(src2, dst2, ew2, n_pad, nch_tot)

    p1p = _sc_prop(x, src2, dst2, norm2, n0, n1)
    p2p = _sc_prop(p1p, src2, dst2, norm2, n0, n1, ((0, n_pad),))

    z128 = jnp.zeros((128, 400), jnp.float32)
    w384 = jnp.concatenate([
        jnp.concatenate([c1s1_W[0], c1s2_W[0], c1s3_W[0] - c1s3_W[2]], axis=1),
        jnp.concatenate([z128, c1s2_W[1], c1s3_W[1]], axis=1),
        jnp.concatenate([z128, z128, 2.0 * c1s3_W[2]], axis=1)], axis=0)
    bias1 = jnp.concatenate([c1s1_b, c1s2_b, c1s3_b])
    pre, stats1 = _tc_l1pre(x, p1p[:n], p1p[n_pad:n_pad + n],
                            p2p[:n], p2p[n_pad:n_pad + n], w384, bias1)

    def pad128(w):
        return jnp.pad(w, ((0, 0), (0, 28)))
    whb = jnp.concatenate([c2s1_W[0], c2s2_W[0], c2s3_W[0] - c2s3_W[2]], axis=1)
    bias2 = jnp.concatenate([c2s1_b, c2s2_b, c2s3_b])
    u1, u2, u3, hb = _tc_l1post(pre, stats1, bn1_g, bn1_b,
                                pad128(c2s2_W[1]), pad128(c2s3_W[1]),
                                pad128(c2s3_W[2]), whb, bias2)

    u = jnp.concatenate([u1, u2, u3], axis=0)
    up = _sc_prop(u, src2, dst2, norm2, n0, n1, ((0,), (n,), (2 * n,)))
    zp = _sc_prop(up, src2, dst2, norm2, n0, n1, ((4 * n_pad, 5 * n_pad),))

    out2, stats2 = _tc_fpre(hb, up[:n], up[n_pad:n_pad + n],
                            up[2 * n_pad:2 * n_pad + n],
                            up[3 * n_pad:3 * n_pad + n],
                            zp[:n], zp[n_pad:n_pad + n])
    return _tc_fpost(out2, stats2, bn2_g, bn2_b, lin_W, lin_b)
